# Initial kernel scaffold; baseline (speedup 1.0000x reference)
import jax
import jax.numpy as jnp
from jax.experimental import pallas as pl


def kernel(inp, emb_0, emb_1, emb_2, emb_3, proj_0, proj_1, proj_2, proj_3):
    raise NotImplementedError("write your pallas kernel here")


# speedup vs baseline: 38.0779x; 38.0779x over previous
import functools

import jax
import jax.numpy as jnp
from jax import lax
from jax.experimental import pallas as pl
from jax.experimental.pallas import tpu as pltpu
from jax.experimental.pallas import tpu_sc as plsc

D_PROJ = 128
EMB_SCALE = float(D_PROJ) ** 0.5
N_VOCAB = 1000000
ROW_BLK = 4000
_BOUND_BLKS = (0, 5, 25, 125, 250)

_NC, _NS = 2, 16
_NW = _NC * _NS
_N_TOK = 819200
_IDX_ROWS = _N_TOK // 128
_RPW = _IDX_ROWS // _NW


def _proj_body(e0, e1, e2, e3, p0, p1, p2, p3, out):
    g = pl.program_id(0)

    def mm(e, p):
        out[...] = lax.dot_general(
            e[...], p[...], (((1,), (1,)), ((), ())),
            preferred_element_type=jnp.float32) * EMB_SCALE

    @pl.when(g < _BOUND_BLKS[1])
    def _():
        mm(e0, p0)

    @pl.when((g >= _BOUND_BLKS[1]) & (g < _BOUND_BLKS[2]))
    def _():
        mm(e1, p1)

    @pl.when((g >= _BOUND_BLKS[2]) & (g < _BOUND_BLKS[3]))
    def _():
        mm(e2, p2)

    @pl.when(g >= _BOUND_BLKS[3])
    def _():
        mm(e3, p3)


def _project(emb0, emb1, emb2, emb3, p0, p1, p2, p3):
    def clampmap(off, nb):
        return lambda g: (jnp.clip(g - off, 0, nb - 1), 0)

    in_specs = [
        pl.BlockSpec((ROW_BLK, 128), clampmap(0, 5)),
        pl.BlockSpec((ROW_BLK, 32), clampmap(5, 20)),
        pl.BlockSpec((ROW_BLK, 8), clampmap(25, 100)),
        pl.BlockSpec((ROW_BLK, 2), clampmap(125, 125)),
        pl.BlockSpec((128, 128), lambda g: (0, 0)),
        pl.BlockSpec((128, 32), lambda g: (0, 0)),
        pl.BlockSpec((128, 8), lambda g: (0, 0)),
        pl.BlockSpec((128, 2), lambda g: (0, 0)),
    ]
    return pl.pallas_call(
        _proj_body,
        grid=(N_VOCAB // ROW_BLK,),
        in_specs=in_specs,
        out_specs=pl.BlockSpec((ROW_BLK, 128), lambda g: (g, 0)),
        out_shape=jax.ShapeDtypeStruct((N_VOCAB, 128), jnp.float32),
    )(emb0, emb1, emb2, emb3, p0, p1, p2, p3)


def _gather(P, idx):
    mesh = plsc.VectorSubcoreMesh(core_axis_name="c", subcore_axis_name="s")

    @functools.partial(
        pl.kernel,
        out_type=jax.ShapeDtypeStruct((_N_TOK, 128), jnp.float32),
        mesh=mesh,
        scratch_types=[
            pltpu.VMEM((_RPW, 128), jnp.int32),
            pltpu.VMEM((128, 128), jnp.float32),
            pltpu.SemaphoreType.DMA,
        ],
    )
    def gk(p_hbm, idx_hbm, out_hbm, idx_v, rows_v, sem):
        wid = lax.axis_index("s") * _NC + lax.axis_index("c")
        row0 = wid * _RPW
        pltpu.sync_copy(idx_hbm.at[pl.ds(row0, _RPW)], idx_v)

        def step(j, carry):
            pltpu.async_copy(p_hbm.at[idx_v.at[j]], rows_v, sem).wait()
            pltpu.sync_copy(rows_v, out_hbm.at[pl.ds((row0 + j) * 128, 128)])
            return carry

        lax.fori_loop(0, _RPW, step, 0)

    return gk(P, idx)


def kernel(inp, emb_0, emb_1, emb_2, emb_3, proj_0, proj_1, proj_2, proj_3):
    P = _project(emb_0, emb_1, emb_2, emb_3, proj_0, proj_1, proj_2, proj_3)
    idx = inp.astype(jnp.int32).reshape(_IDX_ROWS, 128)
    out = _gather(P, idx)
    return out.reshape(inp.shape + (D_PROJ,))


# speedup vs baseline: 43.8381x; 1.1513x over previous
import functools

import jax
import jax.numpy as jnp
from jax import lax
from jax.experimental import pallas as pl
from jax.experimental.pallas import tpu as pltpu
from jax.experimental.pallas import tpu_sc as plsc

D_PROJ = 128
EMB_SCALE = float(D_PROJ) ** 0.5
N_VOCAB = 1000000
ROW_BLK = 4000
_BOUND_BLKS = (0, 5, 25, 125, 250)

_NC, _NS = 2, 16
_NW = _NC * _NS
_N_TOK = 819200
_IDX_ROWS = _N_TOK // 128
_RPW = _IDX_ROWS // _NW


def _proj_body(e0, e1, e2, e3, p0, p1, p2, p3, out):
    g = pl.program_id(0)

    def mm(e, p):
        out[...] = lax.dot_general(
            e[...], p[...], (((1,), (1,)), ((), ())),
            preferred_element_type=jnp.float32) * EMB_SCALE

    @pl.when(g < _BOUND_BLKS[1])
    def _():
        mm(e0, p0)

    @pl.when((g >= _BOUND_BLKS[1]) & (g < _BOUND_BLKS[2]))
    def _():
        mm(e1, p1)

    @pl.when((g >= _BOUND_BLKS[2]) & (g < _BOUND_BLKS[3]))
    def _():
        mm(e2, p2)

    @pl.when(g >= _BOUND_BLKS[3])
    def _():
        mm(e3, p3)


def _project(emb0, emb1, emb2, emb3, p0, p1, p2, p3):
    def clampmap(off, nb):
        return lambda g: (jnp.clip(g - off, 0, nb - 1), 0)

    in_specs = [
        pl.BlockSpec((ROW_BLK, 128), clampmap(0, 5)),
        pl.BlockSpec((ROW_BLK, 32), clampmap(5, 20)),
        pl.BlockSpec((ROW_BLK, 8), clampmap(25, 100)),
        pl.BlockSpec((ROW_BLK, 2), clampmap(125, 125)),
        pl.BlockSpec((128, 128), lambda g: (0, 0)),
        pl.BlockSpec((128, 32), lambda g: (0, 0)),
        pl.BlockSpec((128, 8), lambda g: (0, 0)),
        pl.BlockSpec((128, 2), lambda g: (0, 0)),
    ]
    return pl.pallas_call(
        _proj_body,
        grid=(N_VOCAB // ROW_BLK,),
        in_specs=in_specs,
        out_specs=pl.BlockSpec((ROW_BLK, 128), lambda g: (g, 0)),
        out_shape=jax.ShapeDtypeStruct((N_VOCAB, 128), jnp.float32),
    )(emb0, emb1, emb2, emb3, p0, p1, p2, p3)


_NBUF = 4


def _gather(P, idx):
    mesh = plsc.VectorSubcoreMesh(core_axis_name="c", subcore_axis_name="s")

    @functools.partial(
        pl.kernel,
        out_type=jax.ShapeDtypeStruct((_N_TOK, 128), jnp.float32),
        mesh=mesh,
        scratch_types=[
            pltpu.VMEM((_RPW, 128), jnp.int32),
            pltpu.VMEM((_NBUF, 128, 128), jnp.float32),
            [pltpu.SemaphoreType.DMA] * _NBUF,
            [pltpu.SemaphoreType.DMA] * _NBUF,
        ],
    )
    def gk(p_hbm, idx_hbm, out_hbm, idx_v, rows_v, gsems, wsems):
        wid = lax.axis_index("s") * _NC + lax.axis_index("c")
        row0 = wid * _RPW
        pltpu.sync_copy(idx_hbm.at[pl.ds(row0, _RPW)], idx_v)

        def gs(c, b):
            pltpu.make_async_copy(
                p_hbm.at[idx_v.at[c]], rows_v.at[b], gsems[b]).start()

        def gw(b):
            pltpu.make_async_copy(
                p_hbm.at[idx_v.at[0]], rows_v.at[b], gsems[b]).wait()

        def ws(c, b):
            pltpu.make_async_copy(
                rows_v.at[b], out_hbm.at[pl.ds((row0 + c) * 128, 128)],
                wsems[b]).start()

        def ww(b):
            pltpu.make_async_copy(
                rows_v.at[b], out_hbm.at[pl.ds(row0 * 128, 128)],
                wsems[b]).wait()

        for b in range(_NBUF):
            gs(b, b)

        def step(k, carry):
            base = k * _NBUF
            for b in range(_NBUF):
                gw(b)
                ws(base + b, b)
            for b in range(_NBUF):
                ww(b)
                gs(base + _NBUF + b, b)
            return carry

        n_steady = _RPW // _NBUF - 1
        lax.fori_loop(0, n_steady, step, 0)

        tail = _RPW - _NBUF
        for b in range(_NBUF):
            gw(b)
            ws(tail + b, b)
        for b in range(_NBUF):
            ww(b)

    return gk(P, idx)


def kernel(inp, emb_0, emb_1, emb_2, emb_3, proj_0, proj_1, proj_2, proj_3):
    P = _project(emb_0, emb_1, emb_2, emb_3, proj_0, proj_1, proj_2, proj_3)
    idx = inp.astype(jnp.int32).reshape(_IDX_ROWS, 128)
    out = _gather(P, idx)
    return out.reshape(inp.shape + (D_PROJ,))
